# Initial kernel scaffold; baseline (speedup 1.0000x reference)
import jax
import jax.numpy as jnp
from jax.experimental import pallas as pl


def kernel(x, edge_index, edge_attr, W1, b1, W2, b2, Ws, bs, U1, bu1, gamma, beta, U2, bu2):
    raise NotImplementedError("write your pallas kernel here")


# speedup vs baseline: 4.6336x; 4.6336x over previous
import functools

import jax
import jax.numpy as jnp
from jax import lax
from jax.experimental import pallas as pl
from jax.experimental.pallas import tpu as pltpu
from jax.experimental.pallas import tpu_sc as plsc

N = 10000
E = 320000
D = 128

NC = 2
NS = 16
NW = NC * NS
EPW = E // NW
C = 80
NCH = EPW // C
NQ = (NCH - 1) // 4
NP = 10240
RPT = NP // NS

_mesh = plsc.VectorSubcoreMesh(core_axis_name="c", subcore_axis_name="s")


def _pre_body(x_ref, w1s_ref, w1d_ref, b1_ref, xs_ref, xd_ref):
    x = x_ref[...]
    xs_ref[...] = jnp.dot(x, w1s_ref[...], preferred_element_type=jnp.float32)
    xd_ref[...] = (
        jnp.dot(x, w1d_ref[...], preferred_element_type=jnp.float32) + b1_ref[...]
    )


def _precompute(x, w1s, w1d, b1):
    return pl.pallas_call(
        _pre_body,
        out_shape=(
            jax.ShapeDtypeStruct((N, D), jnp.float32),
            jax.ShapeDtypeStruct((N, D), jnp.float32),
        ),
    )(x, w1s, w1d, b1)


@functools.partial(
    pl.kernel,
    out_type=jax.ShapeDtypeStruct((E, D), jnp.float32),
    mesh=_mesh,
    scratch_types=[
        pltpu.VMEM((NCH, C), jnp.int32),
        pltpu.VMEM((NCH, C), jnp.int32),
        pltpu.VMEM((C, D), jnp.float32),
        pltpu.VMEM((C, D), jnp.float32),
        pltpu.VMEM((C, D), jnp.float32),
        pltpu.VMEM((C, D), jnp.float32),
        pltpu.SemaphoreType.DMA,
        pltpu.SemaphoreType.DMA,
        pltpu.SemaphoreType.DMA,
        pltpu.SemaphoreType.DMA,
    ],
)
def _gather_k(
    xs_hbm, xd_hbm, src_hbm, dst_hbm, g_hbm, idx_s, idx_d, b0, b1, b2, b3, s0, s1, s2, s3
):
    bufs = (b0, b1, b2, b3)
    sems = (s0, s1, s2, s3)
    cid = lax.axis_index("c")
    sid = lax.axis_index("s")
    wid = sid * NC + cid
    base = wid * EPW
    pltpu.sync_copy(src_hbm.at[wid], idx_s)
    pltpu.sync_copy(dst_hbm.at[wid], idx_d)

    def quad(i, _):
        j0 = 4 * i
        dd = [
            pltpu.async_copy(xd_hbm.at[idx_d.at[j0 + t]], bufs[t], sems[t])
            for t in range(4)
        ]
        da = []
        for t in range(4):
            dd[t].wait()
            da.append(
                pltpu.async_copy(xs_hbm.at[idx_s.at[j0 + t]], bufs[t], sems[t], add=True)
            )
        dw = []
        for t in range(4):
            da[t].wait()
            off = pl.multiple_of(base + (j0 + t) * C, C)
            dw.append(pltpu.async_copy(bufs[t], g_hbm.at[pl.ds(off, C)], sems[t]))
        for t in range(4):
            dw[t].wait()
        return 0

    lax.fori_loop(0, NQ, quad, 0)
    j = NCH - 1
    off = pl.multiple_of(base + j * C, C)
    pltpu.async_copy(xd_hbm.at[idx_d.at[j]], b0, s0).wait()
    pltpu.async_copy(xs_hbm.at[idx_s.at[j]], b0, s0, add=True).wait()
    pltpu.sync_copy(b0, g_hbm.at[pl.ds(off, C)])


_BE = 2000


def _edge_body(g_ref, ea_ref, w1e_ref, w2_ref, b2_ref, wst_ref, bs_ref, eo_ref, gm_ref):
    ea = ea_ref[...]
    h = g_ref[...] + jnp.dot(ea, w1e_ref[...], preferred_element_type=jnp.float32)
    h = h * jax.nn.sigmoid(h)
    msg = jnp.dot(h, w2_ref[...], preferred_element_type=jnp.float32) + b2_ref[...]
    msg = msg * jax.nn.sigmoid(msg)
    eo_ref[...] = ea + msg
    ew = jax.nn.sigmoid(
        jnp.sum(msg * wst_ref[...], axis=1, keepdims=True) + bs_ref[0, 0]
    )
    gm_ref[...] = msg * ew


def _edge_mlp(g, ea, w1e, w2, b2, wst, bs):
    blk = lambda: pl.BlockSpec((_BE, D), lambda i: (i, 0))
    wspec = lambda s: pl.BlockSpec(s, lambda i: tuple(0 for _ in s))
    return pl.pallas_call(
        _edge_body,
        grid=(E // _BE,),
        in_specs=[
            blk(),
            blk(),
            wspec((D, D)),
            wspec((D, D)),
            wspec((1, D)),
            wspec((1, D)),
            wspec((1, 1)),
        ],
        out_specs=[blk(), blk()],
        out_shape=(
            jax.ShapeDtypeStruct((E, D), jnp.float32),
            jax.ShapeDtypeStruct((E, D), jnp.float32),
        ),
    )(g, ea, w1e, w2, b2, wst, bs)


@functools.partial(
    pl.kernel,
    out_type=jax.ShapeDtypeStruct((NC, NP, D), jnp.float32),
    mesh=_mesh,
    scratch_types=[
        pltpu.VMEM((NCH, C), jnp.int32),
        pltpu.VMEM((C, D), jnp.float32),
        pltpu.VMEM((C, D), jnp.float32),
        pltpu.VMEM((C, D), jnp.float32),
        pltpu.VMEM_SHARED((NP, D), jnp.float32),
        pltpu.SemaphoreType.DMA,
        pltpu.SemaphoreType.DMA,
        pltpu.SemaphoreType.DMA,
    ],
)
def _scatter_k(gm_hbm, dst_hbm, z_hbm, out_hbm, idx_d, b0, b1, b2, acc, s0, s1, s2):
    bufs = (b0, b1, b2)
    sems = (s0, s1, s2)
    cid = lax.axis_index("c")
    sid = lax.axis_index("s")
    wid = sid * NC + cid
    base = wid * EPW
    rows = pl.ds(sid * RPT, RPT)
    pltpu.sync_copy(z_hbm, acc.at[rows])
    pltpu.sync_copy(dst_hbm.at[wid], idx_d)
    plsc.subcore_barrier()

    def trio(i, _):
        j0 = 3 * i
        dl = []
        for t in range(3):
            off = pl.multiple_of(base + (j0 + t) * C, C)
            dl.append(pltpu.async_copy(gm_hbm.at[pl.ds(off, C)], bufs[t], sems[t]))
        ds_ = []
        for t in range(3):
            dl[t].wait()
            ds_.append(
                pltpu.async_copy(bufs[t], acc.at[idx_d.at[j0 + t]], sems[t], add=True)
            )
        for t in range(3):
            ds_[t].wait()
        return 0

    lax.fori_loop(0, NCH // 3, trio, 0)
    for j in range(3 * (NCH // 3), NCH):
        off = pl.multiple_of(base + j * C, C)
        pltpu.sync_copy(gm_hbm.at[pl.ds(off, C)], b0)
        pltpu.sync_copy(b0, acc.at[idx_d.at[j]], add=True)
    plsc.subcore_barrier()
    pltpu.sync_copy(acc.at[rows], out_hbm.at[cid, rows])


def _node_body(
    p_ref, x_ref, u1_ref, bu1_ref, gamma_ref, beta_ref, u2_ref, bu2_ref, out_ref
):
    x = x_ref[...]
    inp = p_ref[0, :N, :] + p_ref[1, :N, :] + x
    u = jnp.dot(inp, u1_ref[...], preferred_element_type=jnp.float32) + bu1_ref[...]
    u = u * jax.nn.sigmoid(u)
    mean = jnp.mean(u, axis=0, keepdims=True)
    var = jnp.mean((u - mean) * (u - mean), axis=0, keepdims=True)
    un = (u - mean) / jnp.sqrt(var + 1e-5) * gamma_ref[...] + beta_ref[...]
    out_ref[...] = (
        jnp.dot(un, u2_ref[...], preferred_element_type=jnp.float32) + bu2_ref[...] + x
    )


def _node_update(parts, x, u1, bu1, gamma, beta, u2, bu2):
    return pl.pallas_call(
        _node_body,
        out_shape=jax.ShapeDtypeStruct((N, D), jnp.float32),
    )(parts, x, u1, bu1, gamma, beta, u2, bu2)


def kernel(x, edge_index, edge_attr, W1, b1, W2, b2, Ws, bs, U1, bu1, gamma, beta, U2, bu2):
    src3 = edge_index[0].reshape(NW, NCH, C)
    dst3 = edge_index[1].reshape(NW, NCH, C)
    w1s = W1[0:D]
    w1d = W1[D : 2 * D]
    w1e = W1[2 * D :]

    xs, xd = _precompute(x, w1s, w1d, b1.reshape(1, D))
    g = _gather_k(xs, xd, src3, dst3)
    eo, gm = _edge_mlp(
        g, edge_attr, w1e, W2, b2.reshape(1, D), Ws.reshape(1, D), bs.reshape(1, 1)
    )
    zeros = jnp.zeros((RPT, D), jnp.float32)
    parts = _scatter_k(gm, dst3, zeros)
    feat = _node_update(
        parts,
        x,
        U1,
        bu1.reshape(1, D),
        gamma.reshape(1, D),
        beta.reshape(1, D),
        U2,
        bu2.reshape(1, D),
    )
    return feat, eo


# speedup vs baseline: 5.3079x; 1.1455x over previous
import functools

import jax
import jax.numpy as jnp
from jax import lax
from jax.experimental import pallas as pl
from jax.experimental.pallas import tpu as pltpu
from jax.experimental.pallas import tpu_sc as plsc

N = 10000
E = 320000
D = 128

NC = 2
NS = 16
NW = NC * NS
K = 5
ES = E // K
EPW = ES // NW
CG = 80
NCHG = EPW // CG
GD = 8
CS = 40
NCHS = EPW // CS
SD = 6
NP = 10240
RPT = NP // NS

_mesh = plsc.VectorSubcoreMesh(core_axis_name="c", subcore_axis_name="s")


def _pre_body(x_ref, w1s_ref, w1d_ref, b1_ref, xs_ref, xd_ref):
    x = x_ref[...]
    xs_ref[...] = jnp.dot(x, w1s_ref[...], preferred_element_type=jnp.float32)
    xd_ref[...] = (
        jnp.dot(x, w1d_ref[...], preferred_element_type=jnp.float32) + b1_ref[...]
    )


def _precompute(x, w1s, w1d, b1):
    return pl.pallas_call(
        _pre_body,
        out_shape=(
            jax.ShapeDtypeStruct((N, D), jnp.float32),
            jax.ShapeDtypeStruct((N, D), jnp.float32),
        ),
    )(x, w1s, w1d, b1)


def _make_gather(k):
    @functools.partial(
        pl.kernel,
        out_type=jax.ShapeDtypeStruct((ES, D), jnp.float32),
        mesh=_mesh,
        scratch_types=[
            pltpu.VMEM((NCHG, CG), jnp.int32),
            pltpu.VMEM((NCHG, CG), jnp.int32),
        ]
        + [pltpu.VMEM((CG, D), jnp.float32)] * GD
        + [pltpu.SemaphoreType.DMA] * GD,
    )
    def gather_k(xs_hbm, xd_hbm, src_hbm, dst_hbm, g_hbm, idx_s, idx_d, *rest):
        bufs = rest[:GD]
        sems = rest[GD:]
        cid = lax.axis_index("c")
        sid = lax.axis_index("s")
        wid = sid * NC + cid
        base = wid * EPW
        pltpu.sync_copy(src_hbm.at[k, wid], idx_s)
        pltpu.sync_copy(dst_hbm.at[k, wid], idx_d)

        def group(i, _):
            j0 = GD * i
            dd = [
                pltpu.async_copy(xd_hbm.at[idx_d.at[j0 + t]], bufs[t], sems[t])
                for t in range(GD)
            ]
            da = []
            for t in range(GD):
                dd[t].wait()
                da.append(
                    pltpu.async_copy(
                        xs_hbm.at[idx_s.at[j0 + t]], bufs[t], sems[t], add=True
                    )
                )
            dw = []
            for t in range(GD):
                da[t].wait()
                off = pl.multiple_of(base + (j0 + t) * CG, CG)
                dw.append(pltpu.async_copy(bufs[t], g_hbm.at[pl.ds(off, CG)], sems[t]))
            for t in range(GD):
                dw[t].wait()
            return 0

        lax.fori_loop(0, NCHG // GD, group, 0)
        for j in range(GD * (NCHG // GD), NCHG):
            off = pl.multiple_of(base + j * CG, CG)
            pltpu.async_copy(xd_hbm.at[idx_d.at[j]], bufs[0], sems[0]).wait()
            pltpu.async_copy(xs_hbm.at[idx_s.at[j]], bufs[0], sems[0], add=True).wait()
            pltpu.sync_copy(bufs[0], g_hbm.at[pl.ds(off, CG)])

    return gather_k


_BE = 2000
_BPS = ES // _BE


def _edge_body(
    g_ref, ea_ref, w1e_ref, w2_ref, b2_ref, wst_ref, bs_ref, eoin_ref, eo_ref, gm_ref
):
    del eoin_ref
    _edge_math(g_ref, ea_ref, w1e_ref, w2_ref, b2_ref, wst_ref, bs_ref, eo_ref, gm_ref)


def _edge_body_first(
    g_ref, ea_ref, w1e_ref, w2_ref, b2_ref, wst_ref, bs_ref, eo_ref, gm_ref
):
    _edge_math(g_ref, ea_ref, w1e_ref, w2_ref, b2_ref, wst_ref, bs_ref, eo_ref, gm_ref)


def _edge_math(g_ref, ea_ref, w1e_ref, w2_ref, b2_ref, wst_ref, bs_ref, eo_ref, gm_ref):
    ea = ea_ref[...]
    h = g_ref[...] + jnp.dot(ea, w1e_ref[...], preferred_element_type=jnp.float32)
    h = h * jax.nn.sigmoid(h)
    msg = jnp.dot(h, w2_ref[...], preferred_element_type=jnp.float32) + b2_ref[...]
    msg = msg * jax.nn.sigmoid(msg)
    eo_ref[...] = ea + msg
    ew = jax.nn.sigmoid(
        jnp.sum(msg * wst_ref[...], axis=1, keepdims=True) + bs_ref[0, 0]
    )
    gm_ref[...] = msg * ew


def _edge_mlp_slice(k, g, ea, w1e, w2, b2, wst, bs, eo_acc=None):
    sblk = pl.BlockSpec((_BE, D), lambda i: (i, 0))
    fblk = pl.BlockSpec((_BE, D), lambda i, _k=k: (i + _k * _BPS, 0))
    wspec = lambda s: pl.BlockSpec(s, lambda i: tuple(0 for _ in s))
    in_specs = [
        sblk,
        fblk,
        wspec((D, D)),
        wspec((D, D)),
        wspec((1, D)),
        wspec((1, D)),
        wspec((1, 1)),
    ]
    args = [g, ea, w1e, w2, b2, wst, bs]
    aliases = {}
    body = _edge_body_first
    if eo_acc is not None:
        in_specs.append(pl.BlockSpec(memory_space=pl.ANY))
        args.append(eo_acc)
        aliases = {7: 0}
        body = _edge_body
    return pl.pallas_call(
        body,
        grid=(_BPS,),
        in_specs=in_specs,
        out_specs=[fblk, sblk],
        out_shape=(
            jax.ShapeDtypeStruct((E, D), jnp.float32),
            jax.ShapeDtypeStruct((ES, D), jnp.float32),
        ),
        input_output_aliases=aliases,
    )(*args)


def _scatter_call(k, gm, dst_r, init):
    @functools.partial(
        pl.kernel,
        out_type=jax.ShapeDtypeStruct((NC, NP, D), jnp.float32),
        mesh=_mesh,
        scratch_types=[
            pltpu.VMEM((NCHS, CS), jnp.int32),
        ]
        + [pltpu.VMEM((CS, D), jnp.float32)] * SD
        + [pltpu.SemaphoreType.DMA] * SD
        + [pltpu.VMEM_SHARED((NP, D), jnp.float32)],
    )
    def scatter_k(gm_hbm, dst_hbm, init_hbm, out_hbm, idx_d, *rest):
        bufs = rest[:SD]
        sems = rest[SD : 2 * SD]
        acc = rest[2 * SD]
        cid = lax.axis_index("c")
        sid = lax.axis_index("s")
        wid = sid * NC + cid
        base = wid * EPW
        rows = pl.ds(sid * RPT, RPT)
        if k == 0:
            pltpu.sync_copy(init_hbm, acc.at[rows])
        else:
            pltpu.sync_copy(init_hbm.at[cid, rows], acc.at[rows])
        pltpu.sync_copy(dst_hbm.at[k, wid], idx_d)
        plsc.subcore_barrier()

        def group(i, _):
            j0 = SD * i
            dl = []
            for t in range(SD):
                off = pl.multiple_of(base + (j0 + t) * CS, CS)
                dl.append(pltpu.async_copy(gm_hbm.at[pl.ds(off, CS)], bufs[t], sems[t]))
            ds_ = []
            for t in range(SD):
                dl[t].wait()
                ds_.append(
                    pltpu.async_copy(bufs[t], acc.at[idx_d.at[j0 + t]], sems[t], add=True)
                )
            for t in range(SD):
                ds_[t].wait()
            return 0

        lax.fori_loop(0, NCHS // SD, group, 0)
        for j in range(SD * (NCHS // SD), NCHS):
            off = pl.multiple_of(base + j * CS, CS)
            pltpu.sync_copy(gm_hbm.at[pl.ds(off, CS)], bufs[0])
            pltpu.sync_copy(bufs[0], acc.at[idx_d.at[j]], add=True)
        plsc.subcore_barrier()
        pltpu.sync_copy(acc.at[rows], out_hbm.at[cid, rows])

    return scatter_k(gm, dst_r, init)


def _node_body(
    p_ref, x_ref, u1_ref, bu1_ref, gamma_ref, beta_ref, u2_ref, bu2_ref, out_ref
):
    x = x_ref[...]
    inp = p_ref[0, :N, :] + p_ref[1, :N, :] + x
    u = jnp.dot(inp, u1_ref[...], preferred_element_type=jnp.float32) + bu1_ref[...]
    u = u * jax.nn.sigmoid(u)
    mean = jnp.mean(u, axis=0, keepdims=True)
    var = jnp.mean((u - mean) * (u - mean), axis=0, keepdims=True)
    un = (u - mean) / jnp.sqrt(var + 1e-5) * gamma_ref[...] + beta_ref[...]
    out_ref[...] = (
        jnp.dot(un, u2_ref[...], preferred_element_type=jnp.float32) + bu2_ref[...] + x
    )


def _node_update(parts, x, u1, bu1, gamma, beta, u2, bu2):
    return pl.pallas_call(
        _node_body,
        out_shape=jax.ShapeDtypeStruct((N, D), jnp.float32),
    )(parts, x, u1, bu1, gamma, beta, u2, bu2)


def kernel(x, edge_index, edge_attr, W1, b1, W2, b2, Ws, bs, U1, bu1, gamma, beta, U2, bu2):
    src_r = edge_index[0].reshape(K, NW, NCHG, CG)
    dst_rg = edge_index[1].reshape(K, NW, NCHG, CG)
    dst_rs = edge_index[1].reshape(K, NW, NCHS, CS)
    w1s = W1[0:D]
    w1d = W1[D : 2 * D]
    w1e = W1[2 * D :]

    xs, xd = _precompute(x, w1s, w1d, b1.reshape(1, D))

    gs = [_make_gather(k)(xs, xd, src_r, dst_rg) for k in range(K)]

    eo_acc = None
    gms = []
    for k in range(K):
        eo_acc, gm = _edge_mlp_slice(
            k, gs[k], edge_attr, w1e, W2, b2.reshape(1, D),
            Ws.reshape(1, D), bs.reshape(1, 1), eo_acc,
        )
        gms.append(gm)

    part = jnp.zeros((RPT, D), jnp.float32)
    for k in range(K):
        part = _scatter_call(k, gms[k], dst_rs, part)

    feat = _node_update(
        part,
        x,
        U1,
        bu1.reshape(1, D),
        gamma.reshape(1, D),
        beta.reshape(1, D),
        U2,
        bu2.reshape(1, D),
    )
    return feat, eo_acc
